# baseline (speedup 1.0000x reference)
import functools
import jax
import jax.numpy as jnp
from jax.experimental import pallas as pl
from jax.experimental.pallas import tpu as pltpu

N_BLK = 2000


def _sage_combine_kernel(sum_ref, cnt_ref, x_ref, wl_ref, wr_ref, b_ref, o_ref):
    mean = sum_ref[...] / jnp.maximum(cnt_ref[...], 1.0)
    acc = jnp.dot(mean, wl_ref[...], preferred_element_type=jnp.float32)
    acc += jnp.dot(x_ref[...], wr_ref[...], preferred_element_type=jnp.float32)
    acc += b_ref[...]
    o_ref[...] = jnp.maximum(acc, 0.0)


def _sage_combine(s, cnt, x, wl, wr, b):
    n, d = x.shape
    h = wl.shape[1]
    grid = n // N_BLK
    return pl.pallas_call(
        _sage_combine_kernel,
        grid=(grid,),
        in_specs=[
            pl.BlockSpec((N_BLK, d), lambda i: (i, 0)),
            pl.BlockSpec((N_BLK, 1), lambda i: (i, 0)),
            pl.BlockSpec((N_BLK, d), lambda i: (i, 0)),
            pl.BlockSpec((d, h), lambda i: (0, 0)),
            pl.BlockSpec((d, h), lambda i: (0, 0)),
            pl.BlockSpec((1, h), lambda i: (0, 0)),
        ],
        out_specs=pl.BlockSpec((N_BLK, h), lambda i: (i, 0)),
        out_shape=jax.ShapeDtypeStruct((n, h), jnp.float32),
    )(s, cnt, x, wl, wr, b)


def _final_kernel(sum_ref, cnt_ref, hc_ref, wl_ref, wr_ref, b2_ref,
                  wlin_ref, blin_ref, o_ref):
    mean = sum_ref[...] / jnp.maximum(cnt_ref[...], 1.0)
    t = jnp.dot(mean, wl_ref[...], preferred_element_type=jnp.float32)
    t += jnp.dot(hc_ref[...], wr_ref[...], preferred_element_type=jnp.float32)
    t += b2_ref[...]
    o_ref[...] = jnp.dot(t, wlin_ref[...], preferred_element_type=jnp.float32) + blin_ref[...]


def _final(s, cnt, hc, wl, wr, b2, wlin, blin):
    n, h = hc.shape
    o = wlin.shape[1]
    grid = n // N_BLK
    return pl.pallas_call(
        _final_kernel,
        grid=(grid,),
        in_specs=[
            pl.BlockSpec((N_BLK, h), lambda i: (i, 0)),
            pl.BlockSpec((N_BLK, 1), lambda i: (i, 0)),
            pl.BlockSpec((N_BLK, h), lambda i: (i, 0)),
            pl.BlockSpec((h, h), lambda i: (0, 0)),
            pl.BlockSpec((h, h), lambda i: (0, 0)),
            pl.BlockSpec((1, h), lambda i: (0, 0)),
            pl.BlockSpec((h, o), lambda i: (0, 0)),
            pl.BlockSpec((1, o), lambda i: (0, 0)),
        ],
        out_specs=pl.BlockSpec((N_BLK, o), lambda i: (i, 0)),
        out_shape=jax.ShapeDtypeStruct((n, o), jnp.float32),
    )(s, cnt, hc, wl, wr, b2, wlin, blin)


def _seg_sum_cnt(x_src, src, dst, n_dst):
    msgs = jnp.take(x_src, src, axis=0)
    s = jax.ops.segment_sum(msgs, dst, num_segments=n_dst)
    cnt = jax.ops.segment_sum(jnp.ones((src.shape[0],), jnp.float32), dst,
                              num_segments=n_dst)
    return s, cnt


@jax.jit
def kernel(x_customer, x_product, edge_index_buys, edge_index_rev,
           W1_buys_l, W1_buys_r, b1_buys, W1_rev_l, W1_rev_r, b1_rev,
           W2_buys_l, W2_buys_r, b2_buys, W2_rev_l, W2_rev_r, b2_rev,
           W_lin, b_lin):
    n_c = x_customer.shape[0]
    n_p = x_product.shape[0]
    sb = edge_index_buys[0].astype(jnp.int32)
    db = edge_index_buys[1].astype(jnp.int32)
    sr = edge_index_rev[0].astype(jnp.int32)
    dr = edge_index_rev[1].astype(jnp.int32)

    s1p, cnt_b = _seg_sum_cnt(x_customer, sb, db, n_p)
    s1c, cnt_r = _seg_sum_cnt(x_product, sr, dr, n_c)

    h_p = _sage_combine(s1p, cnt_b[:, None], x_product,
                        W1_buys_l, W1_buys_r, b1_buys[None, :])
    h_c = _sage_combine(s1c, cnt_r[:, None], x_customer,
                        W1_rev_l, W1_rev_r, b1_rev[None, :])

    s2c, _ = _seg_sum_cnt(h_p, sr, dr, n_c)

    return _final(s2c, cnt_r[:, None], h_c,
                  W2_rev_l, W2_rev_r, b2_rev[None, :], W_lin, b_lin[None, :])


# speedup vs baseline: 2.4156x; 2.4156x over previous
import functools
import jax
import jax.numpy as jnp
from jax import lax
from jax.experimental import pallas as pl
from jax.experimental.pallas import tpu as pltpu
from jax.experimental.pallas import tpu_sc as plsc

N_NODE = 50000
N_EDGE = 400000
E_PAD = 409600
D = 128
CHUNK = 32
N_CHUNKS = D // CHUNK
N_ACC = 50048
ROWS_PER_TILE = N_ACC // 16
IDX_ROWS = E_PAD // 128
TILE_IDX_ROWS = IDX_ROWS // 16
IDX_ROWS_PER_ITER = 8
GATHER_ROWS = 4
N_ITERS = TILE_IDX_ROWS // IDX_ROWS_PER_ITER
N_BLK = 2000


def _run_pass(table, didx2, sidx2, out_hbm, zeros_hbm, ones_hbm, accum, sidx,
              didx, rows, sem_g, sem_s, s):
    pltpu.sync_copy(zeros_hbm, accum.at[pl.ds(s * ROWS_PER_TILE, ROWS_PER_TILE)])
    if table is None:
        pltpu.sync_copy(ones_hbm, rows)
    plsc.subcore_barrier()

    row_base = s * TILE_IDX_ROWS

    def body(i, carry):
        r0 = row_base + i * IDX_ROWS_PER_ITER
        pltpu.sync_copy(didx2.at[pl.ds(r0, IDX_ROWS_PER_ITER)], didx)
        if table is not None:
            pltpu.sync_copy(sidx2.at[pl.ds(r0, IDX_ROWS_PER_ITER)], sidx)
            for h in range(IDX_ROWS_PER_ITER // GATHER_ROWS):
                descs = [pltpu.async_copy(table.at[sidx.at[h * GATHER_ROWS + j]],
                                          rows.at[j], sem_g)
                         for j in range(GATHER_ROWS)]
                for d in descs:
                    d.wait()
                descs = [pltpu.async_copy(rows.at[j],
                                          accum.at[didx.at[h * GATHER_ROWS + j]],
                                          sem_s, add=True)
                         for j in range(GATHER_ROWS)]
                for d in descs:
                    d.wait()
        else:
            descs = [pltpu.async_copy(rows.at[j % GATHER_ROWS],
                                      accum.at[didx.at[j]], sem_s, add=True)
                     for j in range(IDX_ROWS_PER_ITER)]
            for d in descs:
                d.wait()
        return carry

    lax.fori_loop(0, N_ITERS, body, 0)

    plsc.subcore_barrier()
    pltpu.sync_copy(accum.at[pl.ds(s * ROWS_PER_TILE, ROWS_PER_TILE)],
                    out_hbm.at[pl.ds(s * ROWS_PER_TILE, ROWS_PER_TILE)])
    plsc.subcore_barrier()


def _sc_conv1(xc0, xc1, xc2, xc3, xp0, xp1, xp2, xp3,
              sb2, db2, sr2, dr2, zeros_hbm, ones_hbm,
              ob0, ob1, ob2, ob3, or0, or1, or2, or3, ocb, ocr,
              accum, sidx, didx, rows, sem_g, sem_s):
    s = lax.axis_index("s")
    c = lax.axis_index("c")

    common = dict(zeros_hbm=zeros_hbm, ones_hbm=ones_hbm, accum=accum,
                  sidx=sidx, didx=didx, rows=rows, sem_g=sem_g, sem_s=sem_s,
                  s=s)

    @pl.when(c == 0)
    def _sc0():
        _run_pass(xc0, db2, sb2, ob0, **common)
        _run_pass(xc1, db2, sb2, ob1, **common)
        _run_pass(xp0, dr2, sr2, or0, **common)
        _run_pass(xp1, dr2, sr2, or1, **common)
        _run_pass(None, db2, None, ocb, **common)

    @pl.when(c == 1)
    def _sc1():
        _run_pass(xc2, db2, sb2, ob2, **common)
        _run_pass(xc3, db2, sb2, ob3, **common)
        _run_pass(xp2, dr2, sr2, or2, **common)
        _run_pass(xp3, dr2, sr2, or3, **common)
        _run_pass(None, dr2, None, ocr, **common)


def _sc_conv2(hp0, hp1, hp2, hp3, sr2, dr2, zeros_hbm, ones_hbm,
              o0, o1, o2, o3,
              accum, sidx, didx, rows, sem_g, sem_s):
    s = lax.axis_index("s")
    c = lax.axis_index("c")

    common = dict(zeros_hbm=zeros_hbm, ones_hbm=ones_hbm, accum=accum,
                  sidx=sidx, didx=didx, rows=rows, sem_g=sem_g, sem_s=sem_s,
                  s=s)

    @pl.when(c == 0)
    def _sc0():
        _run_pass(hp0, dr2, sr2, o0, **common)
        _run_pass(hp1, dr2, sr2, o1, **common)

    @pl.when(c == 1)
    def _sc1():
        _run_pass(hp2, dr2, sr2, o2, **common)
        _run_pass(hp3, dr2, sr2, o3, **common)


def _sc_scratch():
    return [
        pltpu.VMEM_SHARED((N_ACC, CHUNK), jnp.float32),
        pltpu.VMEM((IDX_ROWS_PER_ITER, 128), jnp.int32),
        pltpu.VMEM((IDX_ROWS_PER_ITER, 128), jnp.int32),
        pltpu.VMEM((GATHER_ROWS, 128, CHUNK), jnp.float32),
        pltpu.SemaphoreType.DMA,
        pltpu.SemaphoreType.DMA,
    ]


def _seg_sums_conv1(x_customer, x_product, sb2, db2, sr2, dr2):
    mesh = plsc.VectorSubcoreMesh(core_axis_name="c", subcore_axis_name="s")
    f32 = jnp.float32
    out = jax.ShapeDtypeStruct((N_ACC, CHUNK), f32)
    kern = pl.kernel(
        _sc_conv1,
        out_type=[out] * 10,
        mesh=mesh,
        scratch_types=_sc_scratch(),
        compiler_params=pltpu.CompilerParams(use_tc_tiling_on_sc=False),
    )
    xc = [x_customer[:, i * CHUNK:(i + 1) * CHUNK] for i in range(N_CHUNKS)]
    xp = [x_product[:, i * CHUNK:(i + 1) * CHUNK] for i in range(N_CHUNKS)]
    zeros = jnp.zeros((ROWS_PER_TILE, CHUNK), f32)
    ones = jnp.ones((GATHER_ROWS, 128, CHUNK), f32)
    res = kern(*xc, *xp, sb2, db2, sr2, dr2, zeros, ones)
    sum_b = jnp.concatenate([r[:N_NODE] for r in res[0:4]], axis=1)
    sum_r = jnp.concatenate([r[:N_NODE] for r in res[4:8]], axis=1)
    cnt_b = res[8][:N_NODE, :1]
    cnt_r = res[9][:N_NODE, :1]
    return sum_b, sum_r, cnt_b, cnt_r


def _seg_sum_conv2(h_p, sr2, dr2):
    mesh = plsc.VectorSubcoreMesh(core_axis_name="c", subcore_axis_name="s")
    f32 = jnp.float32
    out = jax.ShapeDtypeStruct((N_ACC, CHUNK), f32)
    kern = pl.kernel(
        _sc_conv2,
        out_type=[out] * 4,
        mesh=mesh,
        scratch_types=_sc_scratch(),
        compiler_params=pltpu.CompilerParams(use_tc_tiling_on_sc=False),
    )
    hp = [h_p[:, i * CHUNK:(i + 1) * CHUNK] for i in range(N_CHUNKS)]
    zeros = jnp.zeros((ROWS_PER_TILE, CHUNK), f32)
    ones = jnp.ones((GATHER_ROWS, 128, CHUNK), f32)
    res = kern(*hp, sr2, dr2, zeros, ones)
    return jnp.concatenate([r[:N_NODE] for r in res], axis=1)


def _sage_combine_kernel(sum_ref, cnt_ref, x_ref, wl_ref, wr_ref, b_ref, o_ref):
    mean = sum_ref[...] / jnp.maximum(cnt_ref[...], 1.0)
    acc = jnp.dot(mean, wl_ref[...], preferred_element_type=jnp.float32)
    acc += jnp.dot(x_ref[...], wr_ref[...], preferred_element_type=jnp.float32)
    acc += b_ref[...]
    o_ref[...] = jnp.maximum(acc, 0.0)


def _sage_combine(s, cnt, x, wl, wr, b):
    n, d = x.shape
    h = wl.shape[1]
    grid = n // N_BLK
    return pl.pallas_call(
        _sage_combine_kernel,
        grid=(grid,),
        in_specs=[
            pl.BlockSpec((N_BLK, d), lambda i: (i, 0)),
            pl.BlockSpec((N_BLK, 1), lambda i: (i, 0)),
            pl.BlockSpec((N_BLK, d), lambda i: (i, 0)),
            pl.BlockSpec((d, h), lambda i: (0, 0)),
            pl.BlockSpec((d, h), lambda i: (0, 0)),
            pl.BlockSpec((1, h), lambda i: (0, 0)),
        ],
        out_specs=pl.BlockSpec((N_BLK, h), lambda i: (i, 0)),
        out_shape=jax.ShapeDtypeStruct((n, h), jnp.float32),
    )(s, cnt, x, wl, wr, b)


def _final_kernel(sum_ref, cnt_ref, hc_ref, wl_ref, wr_ref, b2_ref,
                  wlin_ref, blin_ref, o_ref):
    mean = sum_ref[...] / jnp.maximum(cnt_ref[...], 1.0)
    t = jnp.dot(mean, wl_ref[...], preferred_element_type=jnp.float32)
    t += jnp.dot(hc_ref[...], wr_ref[...], preferred_element_type=jnp.float32)
    t += b2_ref[...]
    o_ref[...] = jnp.dot(t, wlin_ref[...], preferred_element_type=jnp.float32) + blin_ref[...]


def _final(s, cnt, hc, wl, wr, b2, wlin, blin):
    n, h = hc.shape
    o = wlin.shape[1]
    grid = n // N_BLK
    return pl.pallas_call(
        _final_kernel,
        grid=(grid,),
        in_specs=[
            pl.BlockSpec((N_BLK, h), lambda i: (i, 0)),
            pl.BlockSpec((N_BLK, 1), lambda i: (i, 0)),
            pl.BlockSpec((N_BLK, h), lambda i: (i, 0)),
            pl.BlockSpec((h, h), lambda i: (0, 0)),
            pl.BlockSpec((h, h), lambda i: (0, 0)),
            pl.BlockSpec((1, h), lambda i: (0, 0)),
            pl.BlockSpec((h, o), lambda i: (0, 0)),
            pl.BlockSpec((1, o), lambda i: (0, 0)),
        ],
        out_specs=pl.BlockSpec((N_BLK, o), lambda i: (i, 0)),
        out_shape=jax.ShapeDtypeStruct((n, o), jnp.float32),
    )(s, cnt, hc, wl, wr, b2, wlin, blin)


@jax.jit
def kernel(x_customer, x_product, edge_index_buys, edge_index_rev,
           W1_buys_l, W1_buys_r, b1_buys, W1_rev_l, W1_rev_r, b1_rev,
           W2_buys_l, W2_buys_r, b2_buys, W2_rev_l, W2_rev_r, b2_rev,
           W_lin, b_lin):
    def _pad(idx, dummy):
        pad = jnp.full((E_PAD - N_EDGE,), dummy, jnp.int32)
        return jnp.concatenate([idx.astype(jnp.int32), pad]).reshape(IDX_ROWS, 128)

    sb2 = _pad(edge_index_buys[0], 0)
    db2 = _pad(edge_index_buys[1], N_NODE)
    sr2 = _pad(edge_index_rev[0], 0)
    dr2 = _pad(edge_index_rev[1], N_NODE)

    sum_b, sum_r, cnt_b, cnt_r = _seg_sums_conv1(
        x_customer, x_product, sb2, db2, sr2, dr2)

    h_p = _sage_combine(sum_b, cnt_b, x_product,
                        W1_buys_l, W1_buys_r, b1_buys[None, :])
    h_c = _sage_combine(sum_r, cnt_r, x_customer,
                        W1_rev_l, W1_rev_r, b1_rev[None, :])

    s2c = _seg_sum_conv2(h_p, sr2, dr2)

    return _final(s2c, cnt_r, h_c,
                  W2_rev_l, W2_rev_r, b2_rev[None, :], W_lin, b_lin[None, :])


# speedup vs baseline: 2.6044x; 1.0782x over previous
import functools
import jax
import jax.numpy as jnp
from jax import lax
from jax.experimental import pallas as pl
from jax.experimental.pallas import tpu as pltpu
from jax.experimental.pallas import tpu_sc as plsc

N_NODE = 50000
N_EDGE = 400000
E_PAD = 409600
D = 128
CHUNK = 32
N_CHUNKS = D // CHUNK
N_ACC = 50048
ROWS_PER_TILE = N_ACC // 16
IDX_ROWS = E_PAD // 128
TILE_IDX_ROWS = IDX_ROWS // 16
IDX_ROWS_PER_ITER = 8
GATHER_ROWS = 4
N_ITERS = TILE_IDX_ROWS // IDX_ROWS_PER_ITER
N_BLK = 2000


def _run_pass(table, didx2, sidx2, out_hbm, zeros_hbm, ones_hbm, accum, sidx,
              didx, rows, sem_g, sem_s, s):
    pltpu.sync_copy(zeros_hbm, accum.at[pl.ds(s * ROWS_PER_TILE, ROWS_PER_TILE)])
    if table is None:
        pltpu.sync_copy(ones_hbm, rows)
    plsc.subcore_barrier()

    row_base = s * TILE_IDX_ROWS

    def body(i, carry):
        r0 = row_base + i * IDX_ROWS_PER_ITER
        pltpu.sync_copy(didx2.at[pl.ds(r0, IDX_ROWS_PER_ITER)], didx)
        if table is not None:
            pltpu.sync_copy(sidx2.at[pl.ds(r0, IDX_ROWS_PER_ITER)], sidx)
            for h in range(IDX_ROWS_PER_ITER // GATHER_ROWS):
                descs = [pltpu.async_copy(table.at[sidx.at[h * GATHER_ROWS + j]],
                                          rows.at[j], sem_g)
                         for j in range(GATHER_ROWS)]
                for d in descs:
                    d.wait()
                descs = [pltpu.async_copy(rows.at[j],
                                          accum.at[didx.at[h * GATHER_ROWS + j]],
                                          sem_s, add=True)
                         for j in range(GATHER_ROWS)]
                for d in descs:
                    d.wait()
        else:
            descs = [pltpu.async_copy(rows.at[j % GATHER_ROWS],
                                      accum.at[didx.at[j]], sem_s, add=True)
                     for j in range(IDX_ROWS_PER_ITER)]
            for d in descs:
                d.wait()
        return carry

    lax.fori_loop(0, N_ITERS, body, 0)

    plsc.subcore_barrier()
    pltpu.sync_copy(accum.at[pl.ds(s * ROWS_PER_TILE, ROWS_PER_TILE)],
                    out_hbm.at[pl.ds(s * ROWS_PER_TILE, ROWS_PER_TILE)])
    plsc.subcore_barrier()


def _sc_conv1(xc0, xc1, xc2, xc3, xp0, xp1, xp2, xp3,
              sb2, db2, sr2, dr2, zeros_hbm, ones_hbm,
              ob0, ob1, ob2, ob3, or0, or1, or2, or3, ocb, ocr,
              accum, sidx, didx, rows, sem_g, sem_s):
    s = lax.axis_index("s")
    c = lax.axis_index("c")

    common = dict(zeros_hbm=zeros_hbm, ones_hbm=ones_hbm, accum=accum,
                  sidx=sidx, didx=didx, rows=rows, sem_g=sem_g, sem_s=sem_s,
                  s=s)

    @pl.when(c == 0)
    def _sc0():
        _run_pass(xc0, db2, sb2, ob0, **common)
        _run_pass(xc1, db2, sb2, ob1, **common)
        _run_pass(xp0, dr2, sr2, or0, **common)
        _run_pass(xp1, dr2, sr2, or1, **common)
        _run_pass(None, db2, None, ocb, **common)

    @pl.when(c == 1)
    def _sc1():
        _run_pass(xc2, db2, sb2, ob2, **common)
        _run_pass(xc3, db2, sb2, ob3, **common)
        _run_pass(xp2, dr2, sr2, or2, **common)
        _run_pass(xp3, dr2, sr2, or3, **common)
        _run_pass(None, dr2, None, ocr, **common)


def _sc_conv2(hp0, hp1, hp2, hp3, sr2, dr2, zeros_hbm, ones_hbm,
              o0, o1, o2, o3,
              accum, sidx, didx, rows, sem_g, sem_s):
    s = lax.axis_index("s")
    c = lax.axis_index("c")

    common = dict(zeros_hbm=zeros_hbm, ones_hbm=ones_hbm, accum=accum,
                  sidx=sidx, didx=didx, rows=rows, sem_g=sem_g, sem_s=sem_s,
                  s=s)

    @pl.when(c == 0)
    def _sc0():
        _run_pass(hp0, dr2, sr2, o0, **common)
        _run_pass(hp1, dr2, sr2, o1, **common)

    @pl.when(c == 1)
    def _sc1():
        _run_pass(hp2, dr2, sr2, o2, **common)
        _run_pass(hp3, dr2, sr2, o3, **common)


def _sc_scratch():
    return [
        pltpu.VMEM_SHARED((N_ACC, CHUNK), jnp.float32),
        pltpu.VMEM((IDX_ROWS_PER_ITER, 128), jnp.int32),
        pltpu.VMEM((IDX_ROWS_PER_ITER, 128), jnp.int32),
        pltpu.VMEM((GATHER_ROWS, 128, CHUNK), jnp.float32),
        pltpu.SemaphoreType.DMA,
        pltpu.SemaphoreType.DMA,
    ]


def _seg_sums_conv1(x_customer, x_product, sb2, db2, sr2, dr2):
    mesh = plsc.VectorSubcoreMesh(core_axis_name="c", subcore_axis_name="s")
    f32 = jnp.float32
    out = jax.ShapeDtypeStruct((N_ACC, CHUNK), f32)
    kern = pl.kernel(
        _sc_conv1,
        out_type=[out] * 10,
        mesh=mesh,
        scratch_types=_sc_scratch(),
        compiler_params=pltpu.CompilerParams(use_tc_tiling_on_sc=False),
    )
    xc = [x_customer[:, i * CHUNK:(i + 1) * CHUNK] for i in range(N_CHUNKS)]
    xp = [x_product[:, i * CHUNK:(i + 1) * CHUNK] for i in range(N_CHUNKS)]
    zeros = jnp.zeros((ROWS_PER_TILE, CHUNK), f32)
    ones = jnp.ones((GATHER_ROWS, 128, CHUNK), f32)
    res = kern(*xc, *xp, sb2, db2, sr2, dr2, zeros, ones)
    return res[0:4], res[4:8], res[8], res[9]


def _seg_sum_conv2(h_p, sr2, dr2):
    mesh = plsc.VectorSubcoreMesh(core_axis_name="c", subcore_axis_name="s")
    f32 = jnp.float32
    out = jax.ShapeDtypeStruct((N_ACC, CHUNK), f32)
    kern = pl.kernel(
        _sc_conv2,
        out_type=[out] * 4,
        mesh=mesh,
        scratch_types=_sc_scratch(),
        compiler_params=pltpu.CompilerParams(use_tc_tiling_on_sc=False),
    )
    hp = list(h_p)
    zeros = jnp.zeros((ROWS_PER_TILE, CHUNK), f32)
    ones = jnp.ones((GATHER_ROWS, 128, CHUNK), f32)
    res = kern(*hp, sr2, dr2, zeros, ones)
    return res


def _sage_combine_kernel(s0, s1, s2, s3, cnt_ref, x_ref, wl_ref, wr_ref,
                         b_ref, o0, o1, o2, o3):
    inv = 1.0 / jnp.maximum(cnt_ref[:, :1], 1.0)
    mean = jnp.concatenate(
        [s0[...] * inv, s1[...] * inv, s2[...] * inv, s3[...] * inv], axis=1)
    acc = jnp.dot(mean, wl_ref[...], preferred_element_type=jnp.float32)
    acc += jnp.dot(x_ref[...], wr_ref[...], preferred_element_type=jnp.float32)
    acc += b_ref[...]
    acc = jnp.maximum(acc, 0.0)
    o0[...] = acc[:, 0:32]
    o1[...] = acc[:, 32:64]
    o2[...] = acc[:, 64:96]
    o3[...] = acc[:, 96:128]


def _sage_combine(s_chunks, cnt, x, wl, wr, b):
    n, d = x.shape
    h = wl.shape[1]
    grid = n // N_BLK
    blk_c = pl.BlockSpec((N_BLK, CHUNK), lambda i: (i, 0))
    return pl.pallas_call(
        _sage_combine_kernel,
        grid=(grid,),
        in_specs=[blk_c] * 4 + [
            blk_c,
            pl.BlockSpec((N_BLK, d), lambda i: (i, 0)),
            pl.BlockSpec((d, h), lambda i: (0, 0)),
            pl.BlockSpec((d, h), lambda i: (0, 0)),
            pl.BlockSpec((1, h), lambda i: (0, 0)),
        ],
        out_specs=[blk_c] * 4,
        out_shape=[jax.ShapeDtypeStruct((n, CHUNK), jnp.float32)] * 4,
    )(*s_chunks, cnt, x, wl, wr, b)


def _final_kernel(s0, s1, s2, s3, cnt_ref, h0, h1, h2, h3, wl_ref, wr_ref,
                  b2_ref, wlin_ref, blin_ref, o_ref):
    inv = 1.0 / jnp.maximum(cnt_ref[:, :1], 1.0)
    mean = jnp.concatenate(
        [s0[...] * inv, s1[...] * inv, s2[...] * inv, s3[...] * inv], axis=1)
    hc = jnp.concatenate([h0[...], h1[...], h2[...], h3[...]], axis=1)
    t = jnp.dot(mean, wl_ref[...], preferred_element_type=jnp.float32)
    t += jnp.dot(hc, wr_ref[...], preferred_element_type=jnp.float32)
    t += b2_ref[...]
    o_ref[...] = jnp.dot(t, wlin_ref[...], preferred_element_type=jnp.float32) + blin_ref[...]


def _final(s_chunks, cnt, hc_chunks, wl, wr, b2, wlin, blin):
    n = N_NODE
    h = wl.shape[0]
    o = wlin.shape[1]
    grid = n // N_BLK
    blk_c = pl.BlockSpec((N_BLK, CHUNK), lambda i: (i, 0))
    return pl.pallas_call(
        _final_kernel,
        grid=(grid,),
        in_specs=[blk_c] * 4 + [blk_c] + [blk_c] * 4 + [
            pl.BlockSpec((h, h), lambda i: (0, 0)),
            pl.BlockSpec((h, h), lambda i: (0, 0)),
            pl.BlockSpec((1, h), lambda i: (0, 0)),
            pl.BlockSpec((h, o), lambda i: (0, 0)),
            pl.BlockSpec((1, o), lambda i: (0, 0)),
        ],
        out_specs=pl.BlockSpec((N_BLK, o), lambda i: (i, 0)),
        out_shape=jax.ShapeDtypeStruct((n, o), jnp.float32),
    )(*s_chunks, cnt, *hc_chunks, wl, wr, b2, wlin, blin)


@jax.jit
def kernel(x_customer, x_product, edge_index_buys, edge_index_rev,
           W1_buys_l, W1_buys_r, b1_buys, W1_rev_l, W1_rev_r, b1_rev,
           W2_buys_l, W2_buys_r, b2_buys, W2_rev_l, W2_rev_r, b2_rev,
           W_lin, b_lin):
    def _pad(idx, dummy):
        pad = jnp.full((E_PAD - N_EDGE,), dummy, jnp.int32)
        return jnp.concatenate([idx.astype(jnp.int32), pad]).reshape(IDX_ROWS, 128)

    sb2 = _pad(edge_index_buys[0], 0)
    db2 = _pad(edge_index_buys[1], N_NODE)
    sr2 = _pad(edge_index_rev[0], 0)
    dr2 = _pad(edge_index_rev[1], N_NODE)

    sum_b, sum_r, cnt_b, cnt_r = _seg_sums_conv1(
        x_customer, x_product, sb2, db2, sr2, dr2)

    hp_chunks = _sage_combine(sum_b, cnt_b, x_product,
                              W1_buys_l, W1_buys_r, b1_buys[None, :])
    hc_chunks = _sage_combine(sum_r, cnt_r, x_customer,
                              W1_rev_l, W1_rev_r, b1_rev[None, :])

    s2c = _seg_sum_conv2(hp_chunks, sr2, dr2)

    return _final(s2c, cnt_r, hc_chunks,
                  W2_rev_l, W2_rev_r, b2_rev[None, :], W_lin, b_lin[None, :])


# speedup vs baseline: 2.8243x; 1.0844x over previous
import functools
import jax
import jax.numpy as jnp
from jax import lax
from jax.experimental import pallas as pl
from jax.experimental.pallas import tpu as pltpu
from jax.experimental.pallas import tpu_sc as plsc

N_NODE = 50000
N_EDGE = 400000
E_PAD = 409600
D = 128
CHUNK = 32
N_CHUNKS = D // CHUNK
N_ACC = 50048
ROWS_PER_TILE = N_ACC // 16
IDX_ROWS = E_PAD // 128
TILE_IDX_ROWS = IDX_ROWS // 16
IDX_ROWS_PER_ITER = 8
GATHER_ROWS = 4
N_ITERS = TILE_IDX_ROWS // IDX_ROWS_PER_ITER
N_BLK = 2000


def _run_pass(table, didx2, sidx2, out_hbm, zeros_hbm, ones_hbm, accum, sidx,
              didx, rows, sem_i, sem_g, sem_s, s):
    pltpu.sync_copy(zeros_hbm, accum.at[pl.ds(s * ROWS_PER_TILE, ROWS_PER_TILE)])
    if table is None:
        pltpu.sync_copy(ones_hbm, rows)
    plsc.subcore_barrier()

    row_base = s * TILE_IDX_ROWS
    R = IDX_ROWS_PER_ITER

    def fetch_idx(it, buf):
        r0 = row_base + it * R
        pltpu.async_copy(didx2.at[pl.ds(r0, R)], didx.at[buf], sem_i)
        if table is not None:
            pltpu.async_copy(sidx2.at[pl.ds(r0, R)], sidx.at[buf], sem_i)

    def wait_idx(it, buf):
        r0 = row_base + it * R
        pltpu.make_async_copy(didx2.at[pl.ds(r0, R)], didx.at[buf], sem_i).wait()
        if table is not None:
            pltpu.make_async_copy(sidx2.at[pl.ds(r0, R)], sidx.at[buf],
                                  sem_i).wait()

    def gather_pair(buf, g, half):
        return [pltpu.async_copy(table.at[sidx.at[buf, 2 * g + j]],
                                 rows.at[2 * half + j], sem_g[half])
                for j in range(2)]

    def scatter_pair(buf, g, half):
        return [pltpu.async_copy(rows.at[2 * half + j],
                                 accum.at[didx.at[buf, 2 * g + j]],
                                 sem_s[half], add=True)
                for j in range(2)]

    def iter_body(it, buf, nxt):
        wait_idx(it, buf)
        if nxt is not None:
            fetch_idx(nxt, 1 - buf)
        if table is not None:
            g0 = gather_pair(buf, 0, 0)
            g1 = gather_pair(buf, 1, 1)
            for d in g0:
                d.wait()
            s0 = scatter_pair(buf, 0, 0)
            for d in g1:
                d.wait()
            s1 = scatter_pair(buf, 1, 1)
            for d in s0:
                d.wait()
            g2 = gather_pair(buf, 2, 0)
            for d in s1:
                d.wait()
            g3 = gather_pair(buf, 3, 1)
            for d in g2:
                d.wait()
            s2 = scatter_pair(buf, 2, 0)
            for d in g3:
                d.wait()
            s3 = scatter_pair(buf, 3, 1)
            for d in s2:
                d.wait()
            for d in s3:
                d.wait()
        else:
            descs = [pltpu.async_copy(rows.at[j % GATHER_ROWS],
                                      accum.at[didx.at[buf, j]],
                                      sem_s[j % 2], add=True)
                     for j in range(R)]
            for d in descs:
                d.wait()

    fetch_idx(0, 0)

    def body(ii, carry):
        iter_body(2 * ii, 0, 2 * ii + 1)
        iter_body(2 * ii + 1, 1, 2 * ii + 2)
        return carry

    lax.fori_loop(0, (N_ITERS - 1) // 2, body, 0)
    iter_body(N_ITERS - 1, 0, None)

    plsc.subcore_barrier()
    pltpu.sync_copy(accum.at[pl.ds(s * ROWS_PER_TILE, ROWS_PER_TILE)],
                    out_hbm.at[pl.ds(s * ROWS_PER_TILE, ROWS_PER_TILE)])
    plsc.subcore_barrier()


def _sc_conv1(xc0, xc1, xc2, xc3, xp0, xp1, xp2, xp3,
              sb2, db2, sr2, dr2, zeros_hbm, ones_hbm,
              ob0, ob1, ob2, ob3, or0, or1, or2, or3, ocb, ocr,
              accum, sidx, didx, rows, sem_i, sem_g, sem_s):
    s = lax.axis_index("s")
    c = lax.axis_index("c")

    common = dict(zeros_hbm=zeros_hbm, ones_hbm=ones_hbm, accum=accum,
                  sidx=sidx, didx=didx, rows=rows, sem_i=sem_i, sem_g=sem_g,
                  sem_s=sem_s, s=s)

    @pl.when(c == 0)
    def _sc0():
        _run_pass(xc0, db2, sb2, ob0, **common)
        _run_pass(xc1, db2, sb2, ob1, **common)
        _run_pass(xp0, dr2, sr2, or0, **common)
        _run_pass(xp1, dr2, sr2, or1, **common)
        _run_pass(None, db2, None, ocb, **common)

    @pl.when(c == 1)
    def _sc1():
        _run_pass(xc2, db2, sb2, ob2, **common)
        _run_pass(xc3, db2, sb2, ob3, **common)
        _run_pass(xp2, dr2, sr2, or2, **common)
        _run_pass(xp3, dr2, sr2, or3, **common)
        _run_pass(None, dr2, None, ocr, **common)


def _sc_conv2(hp0, hp1, hp2, hp3, sr2, dr2, zeros_hbm, ones_hbm,
              o0, o1, o2, o3,
              accum, sidx, didx, rows, sem_i, sem_g, sem_s):
    s = lax.axis_index("s")
    c = lax.axis_index("c")

    common = dict(zeros_hbm=zeros_hbm, ones_hbm=ones_hbm, accum=accum,
                  sidx=sidx, didx=didx, rows=rows, sem_i=sem_i, sem_g=sem_g,
                  sem_s=sem_s, s=s)

    @pl.when(c == 0)
    def _sc0():
        _run_pass(hp0, dr2, sr2, o0, **common)
        _run_pass(hp1, dr2, sr2, o1, **common)

    @pl.when(c == 1)
    def _sc1():
        _run_pass(hp2, dr2, sr2, o2, **common)
        _run_pass(hp3, dr2, sr2, o3, **common)


def _sc_scratch():
    return [
        pltpu.VMEM_SHARED((N_ACC, CHUNK), jnp.float32),
        pltpu.VMEM((2, IDX_ROWS_PER_ITER, 128), jnp.int32),
        pltpu.VMEM((2, IDX_ROWS_PER_ITER, 128), jnp.int32),
        pltpu.VMEM((GATHER_ROWS, 128, CHUNK), jnp.float32),
        pltpu.SemaphoreType.DMA,
        (pltpu.SemaphoreType.DMA, pltpu.SemaphoreType.DMA),
        (pltpu.SemaphoreType.DMA, pltpu.SemaphoreType.DMA),
    ]


def _seg_sums_conv1(x_customer, x_product, sb2, db2, sr2, dr2):
    mesh = plsc.VectorSubcoreMesh(core_axis_name="c", subcore_axis_name="s")
    f32 = jnp.float32
    out = jax.ShapeDtypeStruct((N_ACC, CHUNK), f32)
    kern = pl.kernel(
        _sc_conv1,
        out_type=[out] * 10,
        mesh=mesh,
        scratch_types=_sc_scratch(),
        compiler_params=pltpu.CompilerParams(use_tc_tiling_on_sc=False),
    )
    xc = [x_customer[:, i * CHUNK:(i + 1) * CHUNK] for i in range(N_CHUNKS)]
    xp = [x_product[:, i * CHUNK:(i + 1) * CHUNK] for i in range(N_CHUNKS)]
    zeros = jnp.zeros((ROWS_PER_TILE, CHUNK), f32)
    ones = jnp.ones((GATHER_ROWS, 128, CHUNK), f32)
    res = kern(*xc, *xp, sb2, db2, sr2, dr2, zeros, ones)
    return res[0:4], res[4:8], res[8], res[9]


def _seg_sum_conv2(h_p, sr2, dr2):
    mesh = plsc.VectorSubcoreMesh(core_axis_name="c", subcore_axis_name="s")
    f32 = jnp.float32
    out = jax.ShapeDtypeStruct((N_ACC, CHUNK), f32)
    kern = pl.kernel(
        _sc_conv2,
        out_type=[out] * 4,
        mesh=mesh,
        scratch_types=_sc_scratch(),
        compiler_params=pltpu.CompilerParams(use_tc_tiling_on_sc=False),
    )
    hp = list(h_p)
    zeros = jnp.zeros((ROWS_PER_TILE, CHUNK), f32)
    ones = jnp.ones((GATHER_ROWS, 128, CHUNK), f32)
    res = kern(*hp, sr2, dr2, zeros, ones)
    return res


def _sage_combine_kernel(s0, s1, s2, s3, cnt_ref, x_ref, wl_ref, wr_ref,
                         b_ref, o0, o1, o2, o3):
    inv = 1.0 / jnp.maximum(cnt_ref[:, :1], 1.0)
    mean = jnp.concatenate(
        [s0[...] * inv, s1[...] * inv, s2[...] * inv, s3[...] * inv], axis=1)
    acc = jnp.dot(mean, wl_ref[...], preferred_element_type=jnp.float32)
    acc += jnp.dot(x_ref[...], wr_ref[...], preferred_element_type=jnp.float32)
    acc += b_ref[...]
    acc = jnp.maximum(acc, 0.0)
    o0[...] = acc[:, 0:32]
    o1[...] = acc[:, 32:64]
    o2[...] = acc[:, 64:96]
    o3[...] = acc[:, 96:128]


def _sage_combine(s_chunks, cnt, x, wl, wr, b):
    n, d = x.shape
    h = wl.shape[1]
    grid = n // N_BLK
    blk_c = pl.BlockSpec((N_BLK, CHUNK), lambda i: (i, 0))
    return pl.pallas_call(
        _sage_combine_kernel,
        grid=(grid,),
        in_specs=[blk_c] * 4 + [
            blk_c,
            pl.BlockSpec((N_BLK, d), lambda i: (i, 0)),
            pl.BlockSpec((d, h), lambda i: (0, 0)),
            pl.BlockSpec((d, h), lambda i: (0, 0)),
            pl.BlockSpec((1, h), lambda i: (0, 0)),
        ],
        out_specs=[blk_c] * 4,
        out_shape=[jax.ShapeDtypeStruct((n, CHUNK), jnp.float32)] * 4,
    )(*s_chunks, cnt, x, wl, wr, b)


def _final_kernel(s0, s1, s2, s3, cnt_ref, h0, h1, h2, h3, wl_ref, wr_ref,
                  b2_ref, wlin_ref, blin_ref, o_ref):
    inv = 1.0 / jnp.maximum(cnt_ref[:, :1], 1.0)
    mean = jnp.concatenate(
        [s0[...] * inv, s1[...] * inv, s2[...] * inv, s3[...] * inv], axis=1)
    hc = jnp.concatenate([h0[...], h1[...], h2[...], h3[...]], axis=1)
    t = jnp.dot(mean, wl_ref[...], preferred_element_type=jnp.float32)
    t += jnp.dot(hc, wr_ref[...], preferred_element_type=jnp.float32)
    t += b2_ref[...]
    o_ref[...] = jnp.dot(t, wlin_ref[...], preferred_element_type=jnp.float32) + blin_ref[...]


def _final(s_chunks, cnt, hc_chunks, wl, wr, b2, wlin, blin):
    n = N_NODE
    h = wl.shape[0]
    o = wlin.shape[1]
    grid = n // N_BLK
    blk_c = pl.BlockSpec((N_BLK, CHUNK), lambda i: (i, 0))
    return pl.pallas_call(
        _final_kernel,
        grid=(grid,),
        in_specs=[blk_c] * 4 + [blk_c] + [blk_c] * 4 + [
            pl.BlockSpec((h, h), lambda i: (0, 0)),
            pl.BlockSpec((h, h), lambda i: (0, 0)),
            pl.BlockSpec((1, h), lambda i: (0, 0)),
            pl.BlockSpec((h, o), lambda i: (0, 0)),
            pl.BlockSpec((1, o), lambda i: (0, 0)),
        ],
        out_specs=pl.BlockSpec((N_BLK, o), lambda i: (i, 0)),
        out_shape=jax.ShapeDtypeStruct((n, o), jnp.float32),
    )(*s_chunks, cnt, *hc_chunks, wl, wr, b2, wlin, blin)


@jax.jit
def kernel(x_customer, x_product, edge_index_buys, edge_index_rev,
           W1_buys_l, W1_buys_r, b1_buys, W1_rev_l, W1_rev_r, b1_rev,
           W2_buys_l, W2_buys_r, b2_buys, W2_rev_l, W2_rev_r, b2_rev,
           W_lin, b_lin):
    def _pad(idx, dummy):
        pad = jnp.full((E_PAD - N_EDGE,), dummy, jnp.int32)
        return jnp.concatenate([idx.astype(jnp.int32), pad]).reshape(IDX_ROWS, 128)

    sb2 = _pad(edge_index_buys[0], 0)
    db2 = _pad(edge_index_buys[1], N_NODE)
    sr2 = _pad(edge_index_rev[0], 0)
    dr2 = _pad(edge_index_rev[1], N_NODE)

    sum_b, sum_r, cnt_b, cnt_r = _seg_sums_conv1(
        x_customer, x_product, sb2, db2, sr2, dr2)

    hp_chunks = _sage_combine(sum_b, cnt_b, x_product,
                              W1_buys_l, W1_buys_r, b1_buys[None, :])
    hc_chunks = _sage_combine(sum_r, cnt_r, x_customer,
                              W1_rev_l, W1_rev_r, b1_rev[None, :])

    s2c = _seg_sum_conv2(hp_chunks, sr2, dr2)

    return _final(s2c, cnt_r, hc_chunks,
                  W2_rev_l, W2_rev_r, b2_rev[None, :], W_lin, b_lin[None, :])
